# baseline (speedup 1.0000x reference)
import functools

import jax
import jax.numpy as jnp
from jax import lax
from jax.experimental import pallas as pl
from jax.experimental.pallas import tpu as pltpu
from jax.experimental.pallas import tpu_sc as plsc

NBUF = 4


def _make_kernel(R, C):
    S = 128
    assert C == S * S
    CO = C - S
    NB = S - 1

    info = plsc.get_sparse_core_info()
    NC, NS = info.num_cores, info.num_subcores
    NW = NC * NS
    assert R % NW == 0
    rows_per_w = R // NW
    assert rows_per_w % NBUF == 0 and rows_per_w >= 2 * NBUF

    mesh = plsc.VectorSubcoreMesh(core_axis_name="c", subcore_axis_name="s")

    @functools.partial(
        pl.kernel,
        mesh=mesh,
        out_type=jax.ShapeDtypeStruct((R, CO), jnp.float32),
        scratch_types=(
            [pltpu.VMEM((C,), jnp.float32) for _ in range(NBUF)]
            + [pltpu.VMEM((CO,), jnp.float32) for _ in range(NBUF)]
            + [
                pltpu.SemaphoreType.DMA((NBUF,)),
                pltpu.SemaphoreType.DMA((NBUF,)),
            ]
        ),
    )
    def k(in_hbm, out_hbm, *rest):
        in_bufs = rest[:NBUF]
        out_bufs = rest[NBUF:2 * NBUF]
        sin, sout = rest[2 * NBUF], rest[2 * NBUF + 1]

        wid = lax.axis_index("s") * NC + lax.axis_index("c")
        row0 = wid * rows_per_w

        def start_in(i, slot):
            pltpu.make_async_copy(
                in_hbm.at[row0 + i], in_bufs[slot], sin.at[slot]
            ).start()

        def wait_in(i, slot):
            pltpu.make_async_copy(
                in_hbm.at[row0 + i], in_bufs[slot], sin.at[slot]
            ).wait()

        H1 = 8192

        def out_half(i, slot, half):
            lo, n = (0, H1) if half == 0 else (H1, CO - H1)
            return pltpu.make_async_copy(
                out_bufs[slot].at[pl.ds(lo, n)],
                out_hbm.at[row0 + i, pl.ds(lo, n)],
                sout.at[slot],
            )

        def wait_out(i, slot):
            out_half(i, slot, 0).wait()
            out_half(i, slot, 1).wait()

        def shuffle(i, slot):
            src, dst = in_bufs[slot], out_bufs[slot]

            def move_block(win_i, win_o, off_i, off_o):
                vals = [win_i[pl.ds(off_i + 16 * j, 16)] for j in range(8)]
                for j in range(8):
                    win_o[pl.ds(off_o + 16 * j, 16)] = vals[j]

            def grp(g, carry):
                win_i = src.at[pl.ds(g * 2064, 2064)]
                win_o = dst.at[pl.ds(g * 2048, 2048)]
                for h in range(16):
                    move_block(win_i, win_o, 129 * h + 1, 128 * h)
                return carry
            lax.fori_loop(0, 4, grp, 0)
            out_half(i, slot, 0).start()
            lax.fori_loop(4, 7, grp, 0)
            for b in range(112, NB):
                move_block(src, dst, 129 * b + 1, 128 * b)
            out_half(i, slot, 1).start()

        for s in range(NBUF):
            start_in(s, s)

        def step(g, carry):
            for s in range(NBUF):
                i = g + s
                wait_in(i, s)

                @pl.when(i >= NBUF)
                def _():
                    wait_out(i - NBUF, s)

                shuffle(i, s)

                @pl.when(i + NBUF < rows_per_w)
                def _():
                    start_in(i + NBUF, s)
            return carry

        lax.fori_loop(0, rows_per_w // NBUF, lambda g, c: step(NBUF * g, c), 0)

        for s in range(NBUF):
            wait_out(rows_per_w - NBUF + s, s)

    return k


def kernel(arr):
    R, C = arr.shape
    return _make_kernel(R, C)(arr)


# speedup vs baseline: 1.0730x; 1.0730x over previous
import functools

import jax
import jax.numpy as jnp
from jax import lax
from jax.experimental import pallas as pl
from jax.experimental.pallas import tpu as pltpu
from jax.experimental.pallas import tpu_sc as plsc

NBUF = 4


def _make_kernel(R, C):
    S = 128
    assert C == S * S
    CO = C - S
    NB = S - 1

    info = plsc.get_sparse_core_info()
    NC, NS = info.num_cores, info.num_subcores
    NW = NC * NS
    assert R % NW == 0
    rows_per_w = R // NW
    assert rows_per_w % NBUF == 0 and rows_per_w >= 2 * NBUF

    mesh = plsc.VectorSubcoreMesh(core_axis_name="c", subcore_axis_name="s")

    @functools.partial(
        pl.kernel,
        mesh=mesh,
        out_type=jax.ShapeDtypeStruct((R, CO), jnp.float32),
        scratch_types=(
            [pltpu.VMEM((C,), jnp.float32) for _ in range(NBUF)]
            + [pltpu.VMEM((CO,), jnp.float32) for _ in range(NBUF)]
            + [
                pltpu.SemaphoreType.DMA((NBUF,)),
                pltpu.SemaphoreType.DMA((NBUF,)),
            ]
        ),
    )
    def k(in_hbm, out_hbm, *rest):
        in_bufs = rest[:NBUF]
        out_bufs = rest[NBUF:2 * NBUF]
        sin, sout = rest[2 * NBUF], rest[2 * NBUF + 1]

        wid = lax.axis_index("s") * NC + lax.axis_index("c")
        row0 = wid * rows_per_w

        def start_in(i, slot):
            pltpu.make_async_copy(
                in_hbm.at[row0 + i], in_bufs[slot], sin.at[slot]
            ).start()

        def wait_in(i, slot):
            pltpu.make_async_copy(
                in_hbm.at[row0 + i], in_bufs[slot], sin.at[slot]
            ).wait()

        def start_out(i, slot):
            pltpu.make_async_copy(
                out_bufs[slot], out_hbm.at[row0 + i], sout.at[slot]
            ).start()

        def wait_out(i, slot):
            pltpu.make_async_copy(
                out_bufs[slot], out_hbm.at[row0 + i], sout.at[slot]
            ).wait()

        def shuffle(slot):
            src, dst = in_bufs[slot], out_bufs[slot]

            def move_block(win_i, win_o, off_i, off_o):
                vals = [win_i[pl.ds(off_i + 16 * j, 16)] for j in range(8)]
                for j in range(8):
                    win_o[pl.ds(off_o + 16 * j, 16)] = vals[j]

            def grp(g, carry):
                win_i = src.at[pl.ds(g * 2064, 2064)]
                win_o = dst.at[pl.ds(g * 2048, 2048)]
                for h in range(16):
                    move_block(win_i, win_o, 129 * h + 1, 128 * h)
                return carry
            lax.fori_loop(0, 7, grp, 0)
            for b in range(112, NB):
                move_block(src, dst, 129 * b + 1, 128 * b)

        for s in range(NBUF):
            start_in(s, s)

        def step(g, carry):
            for s in range(NBUF):
                i = g + s
                wait_in(i, s)

                @pl.when(i >= NBUF)
                def _():
                    wait_out(i - NBUF, s)

                shuffle(s)
                start_out(i, s)

                @pl.when(i + NBUF < rows_per_w)
                def _():
                    start_in(i + NBUF, s)
            return carry

        lax.fori_loop(0, rows_per_w // NBUF, lambda g, c: step(NBUF * g, c), 0)

        for s in range(NBUF):
            wait_out(rows_per_w - NBUF + s, s)

    return k


def kernel(arr):
    R, C = arr.shape
    return _make_kernel(R, C)(arr)
